# baseline (speedup 1.0000x reference)
import jax
import jax.numpy as jnp
from jax import lax
from jax.experimental import pallas as pl
from jax.experimental.pallas import tpu as pltpu
from jax.experimental.pallas import tpu_sc as plsc

EMBED_DIM = 64
BATCH = 16384
HIST = 50
VOCAB = 1000000
ROW_W = 2 * EMBED_DIM

NUM_CORES = 2
NUM_SUBCORES = 16
NW = NUM_CORES * NUM_SUBCORES
B_PER_W = BATCH // NW
CHUNK = 128
G_PER_H = B_PER_W // CHUNK
UNITS = HIST * G_PER_H
LANES = 16


def _extract(idx_v, h, g, pidx_s):
    for jg in range(CHUNK // LANES):
        vrow = lax.iota(jnp.int32, LANES) + (g * CHUNK + jg * LANES)
        vcol = jnp.full((LANES,), h, jnp.int32)
        pidx_s[pl.ds(jg * LANES, LANES)] = plsc.load_gather(idx_v, [vrow, vcol])


def _transform(packed_s, stage_s):
    lane = lax.iota(jnp.int32, LANES)

    def jg_body(jg, carry):
        vrow = lane + jg * LANES
        for eg in range(EMBED_DIM // LANES):
            stage_eg = stage_s.at[pl.ds(eg * LANES, LANES)]
            for d in range(LANES):
                rot = (lane + d) & (LANES - 1)
                vals = plsc.load_gather(packed_s, [vrow, rot + (eg * LANES)])
                plsc.store_scatter(stage_eg, [rot, vrow], vals)
        return carry

    lax.fori_loop(0, CHUNK // LANES, jg_body, 0)


def _emb_body(idx_hbm, table_hbm, out_hbm, idx_v, pidx_v, packed_v,
              stage_v, gsem, ssem):
    wid = lax.axis_index("s") * NUM_CORES + lax.axis_index("c")
    b_base = wid * B_PER_W
    pltpu.sync_copy(idx_hbm.at[pl.ds(b_base, B_PER_W)], idx_v)

    def unit_hg(u):
        h = u >> 2
        g = u & (G_PER_H - 1)
        return h, g

    def fire_gather(s):
        pltpu.make_async_copy(
            table_hbm.at[pidx_v.at[s]], packed_v.at[s], gsem.at[s]).start()

    def wait_gather(s):
        pltpu.make_async_copy(
            table_hbm.at[pidx_v.at[s]], packed_v.at[s], gsem.at[s]).wait()

    def store_slice(u, s):
        h, g = unit_hg(u)
        return (stage_v.at[s],
                out_hbm.at[h, :, pl.ds(b_base + g * CHUNK, CHUNK)])

    def fire_store(u, s):
        src, dst = store_slice(u, s)
        pltpu.make_async_copy(src, dst, ssem.at[s]).start()

    def wait_store(u, s):
        src, dst = store_slice(u, s)
        pltpu.make_async_copy(src, dst, ssem.at[s]).wait()

    _extract(idx_v, 0, 0, pidx_v.at[0])
    fire_gather(0)

    def outer(i, carry):
        for s in range(2):
            u = 2 * i + s
            @pl.when(u + 1 < UNITS)
            def _():
                nh, ng = unit_hg(u + 1)
                _extract(idx_v, nh, ng, pidx_v.at[1 - s])
                fire_gather(1 - s)

            wait_gather(s)

            @pl.when(u >= 2)
            def _():
                wait_store(u - 2, s)

            _transform(packed_v.at[s], stage_v.at[s])
            fire_store(u, s)
        return carry

    lax.fori_loop(0, UNITS // 2, outer, 0)

    wait_store(UNITS - 2, 0)
    wait_store(UNITS - 1, 1)


def kernel(token_ids, table):
    idx = token_ids.astype(jnp.int32)
    padded = jnp.pad(table, ((0, 0), (0, ROW_W - EMBED_DIM)))
    f = pl.kernel(
        _emb_body,
        out_type=jax.ShapeDtypeStruct((HIST, EMBED_DIM, BATCH), jnp.float32),
        mesh=plsc.VectorSubcoreMesh(core_axis_name="c", subcore_axis_name="s"),
        scratch_types=[
            pltpu.VMEM((B_PER_W, HIST), jnp.int32),
            pltpu.VMEM((2, CHUNK), jnp.int32),
            pltpu.VMEM((2, CHUNK, ROW_W), jnp.float32),
            pltpu.VMEM((2, EMBED_DIM, CHUNK), jnp.float32),
            pltpu.SemaphoreType.DMA((2,)),
            pltpu.SemaphoreType.DMA((2,)),
        ],
        compiler_params=pltpu.CompilerParams(use_tc_tiling_on_sc=False,
                                             needs_layout_passes=False),
    )
    out = f(idx, padded)
    return jnp.transpose(out, (2, 0, 1))


# speedup vs baseline: 1.2570x; 1.2570x over previous
import jax
import jax.numpy as jnp
from jax import lax
from jax.experimental import pallas as pl
from jax.experimental.pallas import tpu as pltpu
from jax.experimental.pallas import tpu_sc as plsc

EMBED_DIM = 64
BATCH = 16384
HIST = 50
PACKED_ROWS = 500000
ROW_W = 2 * EMBED_DIM

NUM_CORES = 2
NUM_SUBCORES = 16
NW = NUM_CORES * NUM_SUBCORES
B_PER_W = BATCH // NW
CHUNK = 128
G_PER_H = B_PER_W // CHUNK
UNITS = HIST * G_PER_H
LANES = 16


def _extract(idx_v, h, g, pidx_s, sel_s):
    for jg in range(CHUNK // LANES):
        tok = idx_v[h, pl.ds(g * CHUNK + jg * LANES, LANES)]
        pidx_s[pl.ds(jg * LANES, LANES)] = tok >> 1
        sel_s[pl.ds(jg * LANES, LANES)] = (tok & 1) * EMBED_DIM


def _transform(packed_s, sel_s, stage_s):
    lane = lax.iota(jnp.int32, LANES)

    def jg_body(jg, carry):
        vrow = lane + jg * LANES
        vsel = sel_s[pl.ds(jg * LANES, LANES)]
        for eg in range(EMBED_DIM // LANES):
            vsel_eg = vsel + (eg * LANES)
            stage_eg = stage_s.at[pl.ds(eg * LANES, LANES)]
            for d in range(LANES):
                rot = (lane + d) & (LANES - 1)
                vals = plsc.load_gather(packed_s, [vrow, rot + vsel_eg])
                plsc.store_scatter(stage_eg, [rot, vrow], vals)
        return carry

    lax.fori_loop(0, CHUNK // LANES, jg_body, 0)


def _emb_body(idx_hbm, table_hbm, out_hbm, idx_v, pidx_v, sel_v, packed_v,
              stage_v, gsem, ssem):
    wid = lax.axis_index("s") * NUM_CORES + lax.axis_index("c")
    b_base = wid * B_PER_W
    pltpu.sync_copy(idx_hbm.at[:, pl.ds(b_base, B_PER_W)], idx_v)

    def unit_hg(u):
        h = u >> 2
        g = u & (G_PER_H - 1)
        return h, g

    def fire_gather(s):
        pltpu.make_async_copy(
            table_hbm.at[pidx_v.at[s]], packed_v.at[s], gsem.at[s]).start()

    def wait_gather(s):
        pltpu.make_async_copy(
            table_hbm.at[pidx_v.at[s]], packed_v.at[s], gsem.at[s]).wait()

    def store_slice(u, s):
        h, g = unit_hg(u)
        return (stage_v.at[s],
                out_hbm.at[h, :, pl.ds(b_base + g * CHUNK, CHUNK)])

    def fire_store(u, s):
        src, dst = store_slice(u, s)
        pltpu.make_async_copy(src, dst, ssem.at[s]).start()

    def wait_store(u, s):
        src, dst = store_slice(u, s)
        pltpu.make_async_copy(src, dst, ssem.at[s]).wait()

    _extract(idx_v, 0, 0, pidx_v.at[0], sel_v.at[0])
    fire_gather(0)

    def outer(i, carry):
        for s in range(2):
            u = 2 * i + s
            @pl.when(u + 1 < UNITS)
            def _():
                nh, ng = unit_hg(u + 1)
                _extract(idx_v, nh, ng, pidx_v.at[1 - s], sel_v.at[1 - s])
                fire_gather(1 - s)

            wait_gather(s)

            @pl.when(u >= 2)
            def _():
                wait_store(u - 2, s)

            _transform(packed_v.at[s], sel_v.at[s], stage_v.at[s])
            fire_store(u, s)
        return carry

    lax.fori_loop(0, UNITS // 2, outer, 0)

    wait_store(UNITS - 2, 0)
    wait_store(UNITS - 1, 1)


def kernel(token_ids, table):
    idx_t = token_ids.astype(jnp.int32).T
    packed = table.reshape(PACKED_ROWS, ROW_W)
    f = pl.kernel(
        _emb_body,
        out_type=jax.ShapeDtypeStruct((HIST, EMBED_DIM, BATCH), jnp.float32),
        mesh=plsc.VectorSubcoreMesh(core_axis_name="c", subcore_axis_name="s"),
        scratch_types=[
            pltpu.VMEM((HIST, B_PER_W), jnp.int32),
            pltpu.VMEM((2, CHUNK), jnp.int32),
            pltpu.VMEM((2, CHUNK), jnp.int32),
            pltpu.VMEM((2, CHUNK, ROW_W), jnp.float32),
            pltpu.VMEM((2, EMBED_DIM, CHUNK), jnp.float32),
            pltpu.SemaphoreType.DMA((2,)),
            pltpu.SemaphoreType.DMA((2,)),
        ],
        compiler_params=pltpu.CompilerParams(use_tc_tiling_on_sc=True,
                                             needs_layout_passes=False),
    )
    out = f(idx_t, packed)
    return jnp.transpose(out, (2, 0, 1))


# speedup vs baseline: 1.2581x; 1.0009x over previous
import jax
import jax.numpy as jnp
from jax import lax
from jax.experimental import pallas as pl
from jax.experimental.pallas import tpu as pltpu
from jax.experimental.pallas import tpu_sc as plsc

EMBED_DIM = 64
BATCH = 16384
HIST = 50
VOCAB = 1000000
ROW_W = 2 * EMBED_DIM

NUM_CORES = 2
NUM_SUBCORES = 16
NW = NUM_CORES * NUM_SUBCORES
B_PER_W = BATCH // NW
CHUNK = 128
G_PER_H = B_PER_W // CHUNK
UNITS = HIST * G_PER_H
LANES = 16


def _transform(packed_s, stage_s):
    lane = lax.iota(jnp.int32, LANES)

    def jg_body(jg, carry):
        vrow = lane + jg * LANES
        for eg in range(EMBED_DIM // LANES):
            stage_eg = stage_s.at[pl.ds(eg * LANES, LANES)]
            for d in range(LANES):
                rot = (lane + d) & (LANES - 1)
                vals = plsc.load_gather(packed_s, [vrow, rot + (eg * LANES)])
                plsc.store_scatter(stage_eg, [rot, vrow], vals)
        return carry

    lax.fori_loop(0, CHUNK // LANES, jg_body, 0)


def _emb_body(idx_hbm, table_hbm, out_hbm, idx_v, packed_v, stage_v,
              gsem, ssem):
    wid = lax.axis_index("s") * NUM_CORES + lax.axis_index("c")
    b_base = wid * B_PER_W
    pltpu.sync_copy(idx_hbm.at[:, pl.ds(b_base, B_PER_W)], idx_v)

    def unit_hg(u):
        h = u >> 2
        g = u & (G_PER_H - 1)
        return h, g

    def gather_copy(u, s):
        h, g = unit_hg(u)
        idx_ref = idx_v.at[h, pl.ds(g * CHUNK, CHUNK)]
        return pltpu.make_async_copy(
            table_hbm.at[idx_ref], packed_v.at[s], gsem.at[s])

    def store_copy(u, s):
        h, g = unit_hg(u)
        return pltpu.make_async_copy(
            stage_v.at[s],
            out_hbm.at[h, :, pl.ds(b_base + g * CHUNK, CHUNK)], ssem.at[s])

    gather_copy(0, 0).start()

    def outer(i, carry):
        for s in range(2):
            u = 2 * i + s
            @pl.when(u + 1 < UNITS)
            def _():
                gather_copy(u + 1, 1 - s).start()

            gather_copy(u, s).wait()

            @pl.when(u >= 2)
            def _():
                store_copy(u - 2, s).wait()

            _transform(packed_v.at[s], stage_v.at[s])
            store_copy(u, s).start()
        return carry

    lax.fori_loop(0, UNITS // 2, outer, 0)

    store_copy(UNITS - 2, 0).wait()
    store_copy(UNITS - 1, 1).wait()


def kernel(token_ids, table):
    idx_t = token_ids.astype(jnp.int32).T
    padded = jnp.pad(table, ((0, 0), (0, ROW_W - EMBED_DIM)))
    f = pl.kernel(
        _emb_body,
        out_type=jax.ShapeDtypeStruct((HIST, EMBED_DIM, BATCH), jnp.float32),
        mesh=plsc.VectorSubcoreMesh(core_axis_name="c", subcore_axis_name="s"),
        scratch_types=[
            pltpu.VMEM((HIST, B_PER_W), jnp.int32),
            pltpu.VMEM((2, CHUNK, ROW_W), jnp.float32),
            pltpu.VMEM((2, EMBED_DIM, CHUNK), jnp.float32),
            pltpu.SemaphoreType.DMA((2,)),
            pltpu.SemaphoreType.DMA((2,)),
        ],
        compiler_params=pltpu.CompilerParams(use_tc_tiling_on_sc=True,
                                             needs_layout_passes=False),
    )
    out = f(idx_t, padded)
    return jnp.transpose(out, (2, 0, 1))
